# baseline (speedup 1.0000x reference)
import jax
import jax.numpy as jnp
from jax import lax
from jax.experimental import pallas as pl
from jax.experimental.pallas import tpu as pltpu
from jax.experimental.pallas import tpu_sc as plsc

N, D, R = 200000, 100, 100000
DP = 128

NC, NS, L = 2, 16, 16
CR = 5000
NCHUNK = N // (NC * CR)
RPT = 312
TAIL = CR - NS * RPT
ZR = 104
RP = 100096
IPT = RP // NS
NV = IPT // L
B = 128
PADROWS = L
TRASH = IPT + 2 * B


def _sc_body(idx_hbm, recv_hbm, acc_hbm,
             idx_buf, loc_buf, pos_buf, loc2d, gbuf, gbuf2, gsem0, gsem1,
             zbuf, acc):
    c = lax.axis_index("c")
    s = lax.axis_index("s")
    half = N // NC

    pltpu.sync_copy(idx_hbm.at[pl.ds(s * IPT, IPT)], idx_buf)

    zrow = jnp.zeros((L,), jnp.float32)

    def zinit(r, _):
        def zcol(j, _):
            zbuf[r, pl.ds(j * L, L)] = zrow
            return 0
        lax.fori_loop(0, DP // L, zcol, 0)
        return 0
    lax.fori_loop(0, ZR, zinit, 0)

    iota = lax.iota(jnp.int32, L)
    dummy_loc = CR + (s % PADROWS)
    dummy_pos = (s * NC + c) * 97

    def chunk_body(kk, _):
        base = c * half + kk * CR

        def zslice(t, _):
            pltpu.sync_copy(zbuf, acc.at[pl.ds(s * RPT + t * ZR, ZR)])
            return 0
        lax.fori_loop(0, RPT // ZR, zslice, 0)

        @pl.when(s == NS - 1)
        def _():
            pltpu.sync_copy(zbuf.at[pl.ds(0, TAIL)],
                            acc.at[pl.ds(NS * RPT, TAIL)])
        plsc.subcore_barrier()

        def scan_body(vi, cnt):
            iv = idx_buf[pl.ds(vi * L, L)]
            rel = iv - base
            t = rel | (CR - 1 - rel)
            inb = 1 - lax.shift_right_logical(t, 31)
            offs_in = plsc.cumsum(inb) - inb + cnt
            offs = offs_in * inb + TRASH * (1 - inb)
            plsc.store_scatter(loc_buf, [offs], rel)
            pos = s * IPT + vi * L + iota
            plsc.store_scatter(pos_buf, [offs], pos)
            return cnt + jnp.sum(inb)

        cnt = lax.fori_loop(0, NV, scan_body, 0)

        def pad_body(j, _):
            off = cnt + j * L
            loc_buf[pl.ds(off, L)] = jnp.full((L,), dummy_loc, jnp.int32)
            pos_buf[pl.ds(off, L)] = jnp.full((L,), dummy_pos, jnp.int32)
            return 0
        lax.fori_loop(0, B // L, pad_body, 0)

        nb = (cnt + B - 1) // B

        def round_for(gref, gsem, oref, osem, bi):
            pltpu.make_async_copy(
                recv_hbm.at[pos_buf.at[pl.ds(bi * B, B)]], gref, gsem).wait()

            @pl.when(bi + 1 < nb)
            def _():
                pltpu.async_copy(
                    recv_hbm.at[pos_buf.at[pl.ds((bi + 1) * B, B)]],
                    oref, osem)

            def mv(j, _):
                loc2d[0, pl.ds(j * L, L)] = loc_buf[pl.ds(bi * B + j * L, L)]
                return 0
            lax.fori_loop(0, B // L, mv, 0)
            pltpu.sync_copy(gref, acc.at[loc2d.at[0]], add=True)

        @pl.when(nb > 0)
        def _():
            pltpu.async_copy(
                recv_hbm.at[pos_buf.at[pl.ds(0, B)]], gbuf, gsem0)

        def batch_body(bi, _):
            @pl.when(lax.rem(bi, 2) == 0)
            def _():
                round_for(gbuf, gsem0, gbuf2, gsem1, bi)

            @pl.when(lax.rem(bi, 2) == 1)
            def _():
                round_for(gbuf2, gsem1, gbuf, gsem0, bi)
            return 0
        lax.fori_loop(0, nb, batch_body, 0)
        plsc.subcore_barrier()

        pltpu.sync_copy(acc.at[pl.ds(s * RPT, RPT)],
                        acc_hbm.at[pl.ds(base + s * RPT, RPT)])

        @pl.when(s == NS - 1)
        def _():
            pltpu.sync_copy(acc.at[pl.ds(NS * RPT, TAIL)],
                            acc_hbm.at[pl.ds(base + NS * RPT, TAIL)])
        return 0

    lax.fori_loop(0, NCHUNK, chunk_body, 0)


def _sc_scatter(idx_pad, recv_pad):
    mesh = plsc.VectorSubcoreMesh(core_axis_name="c", subcore_axis_name="s")
    fn = pl.kernel(
        _sc_body,
        out_type=jax.ShapeDtypeStruct((N, DP), jnp.float32),
        mesh=mesh,
        compiler_params=pltpu.CompilerParams(needs_layout_passes=False),
        scratch_types=[
            pltpu.VMEM((IPT,), jnp.int32),
            pltpu.VMEM((TRASH + L,), jnp.int32),
            pltpu.VMEM((TRASH + L,), jnp.int32),
            pltpu.VMEM((1, B), jnp.int32),
            pltpu.VMEM((B, DP), jnp.float32),
            pltpu.VMEM((B, DP), jnp.float32),
            pltpu.SemaphoreType.DMA,
            pltpu.SemaphoreType.DMA,
            pltpu.VMEM((ZR, DP), jnp.float32),
            pltpu.VMEM_SHARED((CR + PADROWS, DP), jnp.float32),
        ],
    )
    return fn(idx_pad, recv_pad)


BM = 2048


def _matmul_body(xt_ref, w_ref, b_ref, acc_ref, o_ref):
    mm = lax.dot_general(w_ref[...], xt_ref[...], (((1,), (0,)), ((), ())),
                         preferred_element_type=jnp.float32)
    acct = jnp.transpose(acc_ref[...][:, :D])
    o_ref[...] = mm + b_ref[...][:, None] + acct


def _matmul_add(xt, w, b, acc):
    return pl.pallas_call(
        _matmul_body,
        grid=((N + BM - 1) // BM,),
        in_specs=[
            pl.BlockSpec((D, BM), lambda i: (0, i)),
            pl.BlockSpec((D, D), lambda i: (0, 0)),
            pl.BlockSpec((D,), lambda i: (0,)),
            pl.BlockSpec((BM, DP), lambda i: (i, 0)),
        ],
        out_specs=pl.BlockSpec((D, BM), lambda i: (0, i)),
        out_shape=jax.ShapeDtypeStruct((D, N), jnp.float32),
    )(xt, w, b, acc)


BR = 2048


def _pad_body(rt_ref, o_ref):
    rows = jnp.transpose(rt_ref[...])
    o_ref[...] = jnp.concatenate(
        [rows, jnp.zeros((BR, DP - D), jnp.float32)], axis=1)


def _pad_recv(recvt):
    return pl.pallas_call(
        _pad_body,
        grid=((R + BR - 1) // BR,),
        in_specs=[pl.BlockSpec((D, BR), lambda i: (0, i))],
        out_specs=pl.BlockSpec((BR, DP), lambda i: (i, 0)),
        out_shape=jax.ShapeDtypeStruct((R, DP), jnp.float32),
    )(recvt)


@jax.jit
def _run(x, w, b, from_idx, recv):
    idx = from_idx.astype(jnp.int32)
    idx_pad = jnp.concatenate(
        [idx, jnp.full((RP - R,), N, jnp.int32)])
    recv_pad = _pad_recv(recv.T)
    acc = _sc_scatter(idx_pad, recv_pad)
    outt = _matmul_add(x.T, w, b, acc)
    return outt.T


def kernel(input, W, b, from_idx, to_idx, recv):
    return _run(input, W, b, from_idx, recv)


# speedup vs baseline: 1.1283x; 1.1283x over previous
import jax
import jax.numpy as jnp
from jax import lax
from jax.experimental import pallas as pl
from jax.experimental.pallas import tpu as pltpu
from jax.experimental.pallas import tpu_sc as plsc

N, D, R = 200000, 100, 100000
DP = 128

NC, NS, L = 2, 16, 16
CR = 5000
NCHUNK = N // (NC * CR)
RPT = 312
TAIL = CR - NS * RPT
ZR = 104
RP = 100096
IPT = RP // NS
NV = IPT // L
B = 128
PADROWS = L
TRASH = IPT + 2 * B


def _sc_body(idx_hbm, recv_hbm, acc_hbm,
             idx_buf, loc_buf, pos_buf, loc2d, gbuf, gbuf2, gsem0, gsem1,
             wsem, zbuf, acc):
    c = lax.axis_index("c")
    s = lax.axis_index("s")
    half = N // NC

    pltpu.sync_copy(idx_hbm.at[pl.ds(s * IPT, IPT)], idx_buf)

    zrow = jnp.zeros((L,), jnp.float32)

    def zinit(r, _):
        def zcol(j, _):
            zbuf[r, pl.ds(j * L, L)] = zrow
            return 0
        lax.fori_loop(0, DP // L, zcol, 0)
        return 0
    lax.fori_loop(0, ZR, zinit, 0)

    iota = lax.iota(jnp.int32, L)
    dummy_loc = CR + (s % PADROWS)
    dummy_pos = (s * NC + c) * 97

    def chunk_body(kk, _):
        base = c * half + kk * CR

        def scan_body(vi, cnt):
            iv = idx_buf[pl.ds(vi * L, L)]
            rel = iv - base
            t = rel | (CR - 1 - rel)
            inb = 1 - lax.shift_right_logical(t, 31)
            offs_in = plsc.cumsum(inb) - inb + cnt
            offs = offs_in * inb + TRASH * (1 - inb)
            plsc.store_scatter(loc_buf, [offs], rel)
            pos = s * IPT + vi * L + iota
            plsc.store_scatter(pos_buf, [offs], pos)
            return cnt + jnp.sum(inb)

        cnt = lax.fori_loop(0, NV, scan_body, 0)

        def pad_body(j, _):
            off = cnt + j * L
            loc_buf[pl.ds(off, L)] = jnp.full((L,), dummy_loc, jnp.int32)
            pos_buf[pl.ds(off, L)] = jnp.full((L,), dummy_pos, jnp.int32)
            return 0
        lax.fori_loop(0, B // L, pad_body, 0)

        nb = (cnt + B - 1) // B

        @pl.when(kk > 0)
        def _():
            pbase = base - CR
            pltpu.make_async_copy(
                acc.at[pl.ds(s * RPT, RPT)],
                acc_hbm.at[pl.ds(pbase + s * RPT, RPT)], wsem).wait()

            @pl.when(s == NS - 1)
            def _():
                pltpu.make_async_copy(
                    acc.at[pl.ds(NS * RPT, TAIL)],
                    acc_hbm.at[pl.ds(pbase + NS * RPT, TAIL)], wsem).wait()

        def zslice(t, _):
            pltpu.sync_copy(zbuf, acc.at[pl.ds(s * RPT + t * ZR, ZR)])
            return 0
        lax.fori_loop(0, RPT // ZR, zslice, 0)

        @pl.when(s == NS - 1)
        def _():
            pltpu.sync_copy(zbuf.at[pl.ds(0, TAIL)],
                            acc.at[pl.ds(NS * RPT, TAIL)])
        plsc.subcore_barrier()

        def round_for(gref, gsem, oref, osem, bi):
            pltpu.make_async_copy(
                recv_hbm.at[pos_buf.at[pl.ds(bi * B, B)]], gref, gsem).wait()

            @pl.when(bi + 1 < nb)
            def _():
                pltpu.async_copy(
                    recv_hbm.at[pos_buf.at[pl.ds((bi + 1) * B, B)]],
                    oref, osem)

            def mv(j, _):
                loc2d[0, pl.ds(j * L, L)] = loc_buf[pl.ds(bi * B + j * L, L)]
                return 0
            lax.fori_loop(0, B // L, mv, 0)
            pltpu.sync_copy(gref, acc.at[loc2d.at[0]], add=True)

        @pl.when(nb > 0)
        def _():
            pltpu.async_copy(
                recv_hbm.at[pos_buf.at[pl.ds(0, B)]], gbuf, gsem0)

        def batch_body(bi, _):
            @pl.when(lax.rem(bi, 2) == 0)
            def _():
                round_for(gbuf, gsem0, gbuf2, gsem1, bi)

            @pl.when(lax.rem(bi, 2) == 1)
            def _():
                round_for(gbuf2, gsem1, gbuf, gsem0, bi)
            return 0
        lax.fori_loop(0, nb, batch_body, 0)
        plsc.subcore_barrier()

        pltpu.async_copy(acc.at[pl.ds(s * RPT, RPT)],
                         acc_hbm.at[pl.ds(base + s * RPT, RPT)], wsem)

        @pl.when(s == NS - 1)
        def _():
            pltpu.async_copy(acc.at[pl.ds(NS * RPT, TAIL)],
                             acc_hbm.at[pl.ds(base + NS * RPT, TAIL)], wsem)
        return 0

    lax.fori_loop(0, NCHUNK, chunk_body, 0)

    lbase = c * half + (NCHUNK - 1) * CR
    pltpu.make_async_copy(
        acc.at[pl.ds(s * RPT, RPT)],
        acc_hbm.at[pl.ds(lbase + s * RPT, RPT)], wsem).wait()

    @pl.when(s == NS - 1)
    def _():
        pltpu.make_async_copy(
            acc.at[pl.ds(NS * RPT, TAIL)],
            acc_hbm.at[pl.ds(lbase + NS * RPT, TAIL)], wsem).wait()


def _sc_scatter(idx_pad, recv_pad):
    mesh = plsc.VectorSubcoreMesh(core_axis_name="c", subcore_axis_name="s")
    fn = pl.kernel(
        _sc_body,
        out_type=jax.ShapeDtypeStruct((N, DP), jnp.float32),
        mesh=mesh,
        compiler_params=pltpu.CompilerParams(needs_layout_passes=False),
        scratch_types=[
            pltpu.VMEM((IPT,), jnp.int32),
            pltpu.VMEM((TRASH + L,), jnp.int32),
            pltpu.VMEM((TRASH + L,), jnp.int32),
            pltpu.VMEM((1, B), jnp.int32),
            pltpu.VMEM((B, DP), jnp.float32),
            pltpu.VMEM((B, DP), jnp.float32),
            pltpu.SemaphoreType.DMA,
            pltpu.SemaphoreType.DMA,
            pltpu.SemaphoreType.DMA,
            pltpu.VMEM((ZR, DP), jnp.float32),
            pltpu.VMEM_SHARED((CR + PADROWS, DP), jnp.float32),
        ],
    )
    return fn(idx_pad, recv_pad)


BM = 2048


def _matmul_body(xt_ref, w_ref, b_ref, acc_ref, o_ref):
    mm = lax.dot_general(w_ref[...], xt_ref[...], (((1,), (0,)), ((), ())),
                         preferred_element_type=jnp.float32)
    acct = jnp.transpose(acc_ref[...][:, :D])
    o_ref[...] = mm + b_ref[...][:, None] + acct


def _matmul_add(xt, w, b, acc):
    return pl.pallas_call(
        _matmul_body,
        grid=((N + BM - 1) // BM,),
        in_specs=[
            pl.BlockSpec((D, BM), lambda i: (0, i)),
            pl.BlockSpec((D, D), lambda i: (0, 0)),
            pl.BlockSpec((D,), lambda i: (0,)),
            pl.BlockSpec((BM, DP), lambda i: (i, 0)),
        ],
        out_specs=pl.BlockSpec((D, BM), lambda i: (0, i)),
        out_shape=jax.ShapeDtypeStruct((D, N), jnp.float32),
    )(xt, w, b, acc)


BR = 2048


def _pad_body(rt_ref, o_ref):
    rows = jnp.transpose(rt_ref[...])
    o_ref[...] = jnp.concatenate(
        [rows, jnp.zeros((BR, DP - D), jnp.float32)], axis=1)


def _pad_recv(recvt):
    return pl.pallas_call(
        _pad_body,
        grid=((R + BR - 1) // BR,),
        in_specs=[pl.BlockSpec((D, BR), lambda i: (0, i))],
        out_specs=pl.BlockSpec((BR, DP), lambda i: (i, 0)),
        out_shape=jax.ShapeDtypeStruct((R, DP), jnp.float32),
    )(recvt)


@jax.jit
def _run(x, w, b, from_idx, recv):
    idx = from_idx.astype(jnp.int32)
    idx_pad = jnp.concatenate(
        [idx, jnp.full((RP - R,), N, jnp.int32)])
    recv_pad = _pad_recv(recv.T)
    acc = _sc_scatter(idx_pad, recv_pad)
    outt = _matmul_add(x.T, w, b, acc)
    return outt.T


def kernel(input, W, b, from_idx, to_idx, recv):
    return _run(input, W, b, from_idx, recv)
